# Initial kernel scaffold; baseline (speedup 1.0000x reference)
import jax
import jax.numpy as jnp
from jax.experimental import pallas as pl


def kernel(embedding_matrix, Wq, bq, Wk, bk):
    raise NotImplementedError("write your pallas kernel here")


# speedup vs baseline: 37.7688x; 37.7688x over previous
import functools

import jax
import jax.numpy as jnp
from jax.experimental import pallas as pl
from jax.experimental.pallas import tpu as pltpu

N = 2048
E = 1024
H = 16
DH = 64
TOPK = 64
_INV_SCALE = 1.0 / (DH ** 0.5)
_RB = 256
_PB = 512


def _proj_body(x_ref, wq_ref, bq_ref, wk_ref, bk_ref, q_ref, k_ref):
    x = x_ref[...]
    dn = (((1,), (1,)), ((), ()))
    q_ref[...] = jax.lax.dot_general(
        x, wq_ref[...], dn, preferred_element_type=jnp.float32) + bq_ref[...]
    k_ref[...] = jax.lax.dot_general(
        x, wk_ref[...], dn, preferred_element_type=jnp.float32) + bk_ref[...]


def _attn_body(q_ref, k_ref, o_ref):
    q = q_ref[0]
    k = k_ref[0]
    dn = (((1,), (1,)), ((), ()))
    sim = jax.lax.dot_general(
        q, k, dn, preferred_element_type=jnp.float32) * _INV_SCALE

    ikey = jax.lax.bitcast_convert_type(sim, jnp.int32)
    ukey = jnp.where(ikey < 0, ikey ^ jnp.int32(0x7FFFFFFF), ikey)

    def body(_, carry):
        t, bit = carry
        cand = t + bit
        cnt = jnp.sum((ukey >= cand).astype(jnp.int32), axis=1, keepdims=True)
        return jnp.where(cnt >= TOPK, cand, t), bit >> 1

    cnt0 = jnp.sum((ukey >= 0).astype(jnp.int32), axis=1, keepdims=True)
    t0 = jnp.where(cnt0 >= TOPK, jnp.int32(0), jnp.int32(-2147483648))
    t0 = jnp.broadcast_to(t0, (q.shape[0], 1))
    t, _ = jax.lax.fori_loop(0, 31, body, (t0, jnp.int32(1 << 30)))

    o_ref[0] = jnp.where(ukey >= t, sim, 0.0)


@jax.jit
def kernel(embedding_matrix, Wq, bq, Wk, bk):
    x = embedding_matrix
    q, k = pl.pallas_call(
        _proj_body,
        grid=(N // _PB,),
        in_specs=[
            pl.BlockSpec((_PB, E), lambda i: (i, 0)),
            pl.BlockSpec((E, E), lambda i: (0, 0)),
            pl.BlockSpec((1, E), lambda i: (0, 0)),
            pl.BlockSpec((E, E), lambda i: (0, 0)),
            pl.BlockSpec((1, E), lambda i: (0, 0)),
        ],
        out_specs=[
            pl.BlockSpec((_PB, E), lambda i: (i, 0)),
            pl.BlockSpec((_PB, E), lambda i: (i, 0)),
        ],
        out_shape=[
            jax.ShapeDtypeStruct((N, E), jnp.float32),
            jax.ShapeDtypeStruct((N, E), jnp.float32),
        ],
    )(x, Wq, bq.reshape(1, E), Wk, bk.reshape(1, E))

    qh = q.reshape(N, H, DH).transpose(1, 0, 2)
    kh = k.reshape(N, H, DH).transpose(1, 0, 2)

    out = pl.pallas_call(
        _attn_body,
        grid=(H, N // _RB),
        in_specs=[
            pl.BlockSpec((1, _RB, DH), lambda h, i: (h, i, 0)),
            pl.BlockSpec((1, N, DH), lambda h, i: (h, 0, 0)),
        ],
        out_specs=pl.BlockSpec((1, _RB, N), lambda h, i: (h, i, 0)),
        out_shape=jax.ShapeDtypeStruct((H, N, N), jnp.float32),
    )(qh, kh)
    return out


# speedup vs baseline: 40.9562x; 1.0844x over previous
import functools

import jax
import jax.numpy as jnp
from jax.experimental import pallas as pl
from jax.experimental.pallas import tpu as pltpu

N = 2048
E = 1024
H = 16
DH = 64
TOPK = 64
_INV_SCALE = 1.0 / (DH ** 0.5)
_RB = 256
_PB = 512


def _proj_body(x_ref, wq_ref, bq_ref, wk_ref, bk_ref, q_ref, k_ref):
    x = x_ref[...]
    dn = (((1,), (1,)), ((), ()))
    q_ref[...] = jax.lax.dot_general(
        x, wq_ref[...], dn, preferred_element_type=jnp.float32) + bq_ref[...]
    k_ref[...] = jax.lax.dot_general(
        x, wk_ref[...], dn, preferred_element_type=jnp.float32) + bk_ref[...]


def _attn_body(q_ref, k_ref, o_ref):
    q = q_ref[0]
    k = k_ref[0]
    dn = (((1,), (1,)), ((), ()))
    sim = jax.lax.dot_general(
        q, k, dn, preferred_element_type=jnp.float32) * _INV_SCALE

    ikey = jax.lax.bitcast_convert_type(sim, jnp.int32)
    ukey = jnp.where(ikey < 0, ikey ^ jnp.int32(0x7FFFFFFF), ikey)

    rb = q.shape[0]

    def cond(carry):
        _, cnt_t, bit = carry
        return jnp.logical_and(bit > 0, jnp.any(cnt_t != TOPK))

    def body(carry):
        t, cnt_t, bit = carry
        cand = t + bit
        cnt = jnp.sum((ukey >= cand).astype(jnp.int32), axis=1, keepdims=True)
        take = cnt >= TOPK
        return (jnp.where(take, cand, t), jnp.where(take, cnt, cnt_t),
                bit >> 1)

    cnt0 = jnp.sum((ukey >= 0).astype(jnp.int32), axis=1, keepdims=True)
    pos = cnt0 >= TOPK
    t0 = jnp.where(pos, jnp.int32(0), jnp.int32(-2147483648))
    t0 = jnp.broadcast_to(t0, (rb, 1))
    cnt_t0 = jnp.where(pos, cnt0, jnp.full((rb, 1), jnp.int32(N)))
    t, _, _ = jax.lax.while_loop(cond, body, (t0, cnt_t0, jnp.int32(1 << 30)))

    o_ref[0] = jnp.where(ukey >= t, sim, 0.0)


@jax.jit
def kernel(embedding_matrix, Wq, bq, Wk, bk):
    x = embedding_matrix
    q, k = pl.pallas_call(
        _proj_body,
        grid=(N // _PB,),
        in_specs=[
            pl.BlockSpec((_PB, E), lambda i: (i, 0)),
            pl.BlockSpec((E, E), lambda i: (0, 0)),
            pl.BlockSpec((1, E), lambda i: (0, 0)),
            pl.BlockSpec((E, E), lambda i: (0, 0)),
            pl.BlockSpec((1, E), lambda i: (0, 0)),
        ],
        out_specs=[
            pl.BlockSpec((_PB, E), lambda i: (i, 0)),
            pl.BlockSpec((_PB, E), lambda i: (i, 0)),
        ],
        out_shape=[
            jax.ShapeDtypeStruct((N, E), jnp.float32),
            jax.ShapeDtypeStruct((N, E), jnp.float32),
        ],
    )(x, Wq, bq.reshape(1, E), Wk, bk.reshape(1, E))

    qh = q.reshape(N, H, DH).transpose(1, 0, 2)
    kh = k.reshape(N, H, DH).transpose(1, 0, 2)

    out = pl.pallas_call(
        _attn_body,
        grid=(H, N // _RB),
        in_specs=[
            pl.BlockSpec((1, _RB, DH), lambda h, i: (h, i, 0)),
            pl.BlockSpec((1, N, DH), lambda h, i: (h, 0, 0)),
        ],
        out_specs=pl.BlockSpec((1, _RB, N), lambda h, i: (h, i, 0)),
        out_shape=jax.ShapeDtypeStruct((H, N, N), jnp.float32),
    )(qh, kh)
    return out
